# Initial kernel scaffold; baseline (speedup 1.0000x reference)
import jax
import jax.numpy as jnp
from jax.experimental import pallas as pl


def kernel(xyz_0, xyz_1, xyz_2, xyz_3, xyz_4, normal_0, normal_1, normal_2, normal_3, normal_4, params):
    raise NotImplementedError("write your pallas kernel here")


# speedup vs baseline: 1.0082x; 1.0082x over previous
import functools

import jax
import jax.numpy as jnp
from jax.experimental import pallas as pl
from jax.experimental.pallas import tpu as pltpu

K = 16
C_MID = 4


def _stem_body(x_ref, w0_ref, b0_ref, w1_ref, b1_ref, w2_ref, b2_ref, o_ref):
    x = x_ref[...]
    h = jax.nn.relu(jnp.dot(x, w0_ref[...], preferred_element_type=jnp.float32) + b0_ref[...])
    h = jax.nn.relu(jnp.dot(h, w1_ref[...], preferred_element_type=jnp.float32) + b1_ref[...])
    h = jax.nn.relu(jnp.dot(h, w2_ref[...], preferred_element_type=jnp.float32) + b2_ref[...])
    o_ref[...] = h


def _stem(xyz0, pw):
    B, N, _ = xyz0.shape
    (w0, b0), (w1, b1), (w2, b2) = pw
    x = xyz0.reshape(B * N, 3)
    out = pl.pallas_call(
        _stem_body,
        out_shape=jax.ShapeDtypeStruct((B * N, 16), jnp.float32),
    )(x, w0, b0, w1, b1, w2, b2)
    return out.reshape(B, N, 16)


def _knn(q, r, k):
    d = jnp.sum(q * q, -1, keepdims=True) - 2.0 * jnp.einsum('bsd,bnd->bsn', q, r) + jnp.sum(r * r, -1)[:, None, :]
    return jax.lax.top_k(-d, k)[1]


def _gather(x, idx):
    return jax.vmap(lambda xb, ib: xb[ib])(x, idx)


def _pointconv(p, xyz, feats, new_xyz, nn_i):
    B, S, _ = new_xyz.shape
    gx = _gather(xyz, nn_i) - new_xyz[:, :, None, :]
    gf = _gather(feats, nn_i)
    w = jax.nn.relu(gx @ p["w1"][0] + p["w1"][1])
    w = jax.nn.relu(w @ p["w2"][0] + p["w2"][1])
    h = jnp.einsum('bskc,bskm->bscm', gf, w).reshape(B, S, -1)
    return jax.nn.relu(h @ p["wl"][0] + p["wl"][1])


def kernel(xyz_0, xyz_1, xyz_2, xyz_3, xyz_4, normal_0, normal_1, normal_2, normal_3, normal_4, params):
    xyzs = [xyz_0, xyz_1, xyz_2, xyz_3, xyz_4]
    feats = _stem(xyzs[0], params["pw"])
    for i, lp in enumerate(params["layers"]):
        xyz, new_xyz = xyzs[i], xyzs[i + 1]
        idx = _knn(new_xyz, xyz, K)
        feats = _pointconv(lp["interp"], xyz, feats, new_xyz, idx)
        nn_idx = _knn(new_xyz, new_xyz, K)
        h = jax.nn.relu(feats @ lp["down"][0] + lp["down"][1])
        h = _pointconv(lp["pc"], new_xyz, h, new_xyz, nn_idx)
        h = h @ lp["up"][0] + lp["up"][1]
        feats = jax.nn.relu(feats + h)
    local = feats
    g = jnp.concatenate([xyzs[-1], feats], axis=-1)
    (W1, b1), (W2, b2) = params["global"]
    g = jax.nn.relu(g @ W1 + b1)
    g = g @ W2 + b2
    g = jnp.max(g, axis=1, keepdims=True)
    return local, g


# speedup vs baseline: 20.9782x; 20.8084x over previous
import functools

import jax
import jax.numpy as jnp
from jax import lax
from jax.experimental import pallas as pl
from jax.experimental.pallas import tpu as pltpu
from jax.experimental.pallas import tpu_sc as plsc

K = 16


def _stem_body(x_ref, w0_ref, b0_ref, w1_ref, b1_ref, w2_ref, b2_ref, o_ref):
    x = x_ref[...]
    h = jax.nn.relu(jnp.dot(x, w0_ref[...], preferred_element_type=jnp.float32) + b0_ref[...])
    h = jax.nn.relu(jnp.dot(h, w1_ref[...], preferred_element_type=jnp.float32) + b1_ref[...])
    h = jax.nn.relu(jnp.dot(h, w2_ref[...], preferred_element_type=jnp.float32) + b2_ref[...])
    o_ref[...] = h


def _stem(xyz0, pw):
    B, N, _ = xyz0.shape
    (w0, b0), (w1, b1), (w2, b2) = pw
    x = xyz0.reshape(B * N, 3)
    out = pl.pallas_call(
        _stem_body,
        out_shape=jax.ShapeDtypeStruct((B * N, 16), jnp.float32),
    )(x, w0, b0, w1, b1, w2, b2)
    return out.reshape(B, N, 16)


def _ce(v, ix, i, j):
    p = v[i] > v[j]
    vi = jnp.where(p, v[j], v[i])
    vj = jnp.where(p, v[i], v[j])
    xi = jnp.where(p, ix[j], ix[i])
    xj = jnp.where(p, ix[i], ix[j])
    v[i], v[j], ix[i], ix[j] = vi, vj, xi, xj


def _bitonic_sort16(v, ix):
    for k in (2, 4, 8, 16):
        step = k // 2
        while step >= 1:
            for i in range(16):
                l = i ^ step
                if l > i:
                    if (i & k) == 0:
                        _ce(v, ix, i, l)
                    else:
                        _ce(v, ix, l, i)
            step //= 2


def _bitonic_merge16(v, ix):
    for step in (8, 4, 2, 1):
        for i in range(16):
            l = i ^ step
            if l > i:
                _ce(v, ix, i, l)


def _knn_body(qT_ref, r_ref, o_ref, *, N, bs, n_off_stride):
    b = pl.program_id(0)
    qT = qT_ref[0]
    r = r_ref[0]
    rr = jnp.sum(r * r, axis=1, keepdims=True)
    qq = jnp.sum(qT * qT, axis=0, keepdims=True)
    dT = rr - 2.0 * jnp.dot(r, qT, preferred_element_type=jnp.float32) + qq

    G = N // 16
    base = b * n_off_stride
    giota = lax.broadcasted_iota(jnp.int32, (G, bs), 0)
    v = [dT[j * G:(j + 1) * G, :] for j in range(16)]
    ix = [giota + (j * G + base) for j in range(16)]

    _bitonic_sort16(v, ix)

    g = G
    while g > 1:
        h = g // 2
        a_v = [v[j][:h, :] for j in range(16)]
        a_x = [ix[j][:h, :] for j in range(16)]
        b_v = [v[j][h:, :] for j in range(16)]
        b_x = [ix[j][h:, :] for j in range(16)]
        nv, nx = [], []
        for j in range(16):
            bv, bx = b_v[15 - j], b_x[15 - j]
            p = a_v[j] > bv
            nv.append(jnp.where(p, bv, a_v[j]))
            nx.append(jnp.where(p, bx, a_x[j]))
        v, ix = nv, nx
        _bitonic_merge16(v, ix)
        g = h

    out = jnp.concatenate(ix, axis=0)
    o_ref[0] = out.T.astype(jnp.int32)


def _knn(q, r, flat_offset_stride, bs=256):
    B, S, _ = q.shape
    N = r.shape[1]
    qT = q.transpose(0, 2, 1)
    body = functools.partial(_knn_body, N=N, bs=bs, n_off_stride=flat_offset_stride)
    out = pl.pallas_call(
        body,
        grid=(B, S // bs),
        in_specs=[
            pl.BlockSpec((1, 3, bs), lambda b, s: (b, 0, s)),
            pl.BlockSpec((1, N, 3), lambda b, s: (b, 0, 0)),
        ],
        out_specs=pl.BlockSpec((1, bs, 16), lambda b, s: (b, s, 0)),
        out_shape=jax.ShapeDtypeStruct((B, S, 16), jnp.int32),
    )(qT, r)
    return out


def _sc_gather(table, idx, chunk=128):
    R, D = table.shape
    M = idx.shape[0]
    NW = 32
    assert M % NW == 0
    b_per_w = M // NW
    chunk = min(chunk, b_per_w)
    assert b_per_w % chunk == 0
    n_ch = b_per_w // chunk
    mesh = plsc.VectorSubcoreMesh(core_axis_name="c", subcore_axis_name="s")

    def body(table_hbm, idx_hbm, out_hbm, idx_v, rows_v, sem):
        wid = lax.axis_index("s") * 2 + lax.axis_index("c")
        base = wid * b_per_w

        def step(ch, _):
            off = base + ch * chunk
            pltpu.sync_copy(idx_hbm.at[pl.ds(off, chunk)], idx_v)
            pltpu.async_copy(table_hbm.at[idx_v], rows_v, sem).wait()
            pltpu.sync_copy(rows_v, out_hbm.at[pl.ds(off, chunk)])
            return _

        lax.fori_loop(0, n_ch, step, 0, unroll=False)

    f = pl.kernel(
        body,
        out_type=jax.ShapeDtypeStruct((M, D), jnp.float32),
        mesh=mesh,
        compiler_params=pltpu.CompilerParams(use_tc_tiling_on_sc=False),
        scratch_types=[
            pltpu.VMEM((chunk,), jnp.int32),
            pltpu.VMEM((chunk, D), jnp.float32),
            pltpu.SemaphoreType.DMA,
        ],
    )
    return f(table, idx)


def _pad_cols(x, D):
    pad = D - x.shape[-1]
    if pad:
        x = jnp.concatenate([x, jnp.zeros(x.shape[:-1] + (pad,), x.dtype)], -1)
    return x


def _round16(n):
    return (n + 15) // 16 * 16


def _wnet(gx, w1, b1, w2, b2):
    w = jax.nn.relu(jnp.dot(gx, w1, preferred_element_type=jnp.float32) + b1)
    return jax.nn.relu(jnp.dot(w, w2, preferred_element_type=jnp.float32) + b2)


def _agg(rows, nx, w1, b1, w2, b2, C, bs):
    gx = rows[:, C:C + 3] - jnp.broadcast_to(
        nx[:, None, :], (bs, K, 3)).reshape(bs * K, 3)
    w = _wnet(gx, w1, b1, w2, b2)
    gf3 = rows[:, :C].reshape(bs, K, C)
    w3 = w.reshape(bs, K, 4)
    hs = [jnp.sum(gf3 * w3[:, :, m:m + 1], axis=1) for m in range(4)]
    return jnp.concatenate(hs, axis=-1)


def _conv_interp_body(rows_ref, nx_ref, w1_ref, b1_ref, w2_ref, b2_ref,
                      wl_ref, bl_ref, wd_ref, bd_ref, f1_ref, hd_ref, *, C, bs):
    h = _agg(rows_ref[...], nx_ref[...], w1_ref[...], b1_ref[...],
             w2_ref[...], b2_ref[...], C, bs)
    f1 = jax.nn.relu(jnp.dot(h, wl_ref[...], preferred_element_type=jnp.float32) + bl_ref[...])
    f1_ref[...] = f1
    hd_ref[...] = jax.nn.relu(jnp.dot(f1, wd_ref[...], preferred_element_type=jnp.float32) + bd_ref[...])


def _conv_pc_body(rows_ref, nx_ref, f1_ref, w1_ref, b1_ref, w2_ref, b2_ref,
                  wl_ref, bl_ref, wu_ref, bu_ref, o_ref, *, C, bs):
    h = _agg(rows_ref[...], nx_ref[...], w1_ref[...], b1_ref[...],
             w2_ref[...], b2_ref[...], C, bs)
    hp = jax.nn.relu(jnp.dot(h, wl_ref[...], preferred_element_type=jnp.float32) + bl_ref[...])
    up = jnp.dot(hp, wu_ref[...], preferred_element_type=jnp.float32) + bu_ref[...]
    o_ref[...] = jax.nn.relu(f1_ref[...] + up)


def _perm_wl(wl, C):
    c_out = wl.shape[1]
    return wl.reshape(C, 4, c_out).transpose(1, 0, 2).reshape(4 * C, c_out)


def _full(shape):
    return pl.BlockSpec(shape, lambda g: tuple(0 for _ in shape))


def _conv_interp(rows, nx_flat, p, down, C, bs=256):
    BS = nx_flat.shape[0]
    D = rows.shape[1]
    (w1, b1), (w2, b2), (wl, bl) = p["w1"], p["w2"], p["wl"]
    wd, bd = down
    c_out, mid = wl.shape[1], wd.shape[1]
    wlp = _perm_wl(wl, C)
    body = functools.partial(_conv_interp_body, C=C, bs=bs)
    f1, hd = pl.pallas_call(
        body,
        grid=(BS // bs,),
        in_specs=[
            pl.BlockSpec((bs * K, D), lambda g: (g, 0)),
            pl.BlockSpec((bs, 3), lambda g: (g, 0)),
            _full(w1.shape), _full(b1.shape), _full(w2.shape), _full(b2.shape),
            _full(wlp.shape), _full(bl.shape), _full(wd.shape), _full(bd.shape),
        ],
        out_specs=[
            pl.BlockSpec((bs, c_out), lambda g: (g, 0)),
            pl.BlockSpec((bs, mid), lambda g: (g, 0)),
        ],
        out_shape=[
            jax.ShapeDtypeStruct((BS, c_out), jnp.float32),
            jax.ShapeDtypeStruct((BS, mid), jnp.float32),
        ],
    )(rows, nx_flat, w1, b1, w2, b2, wlp, bl, wd, bd)
    return f1, hd


def _conv_pc(rows, nx_flat, f1, p, up, C, bs=256):
    BS = nx_flat.shape[0]
    D = rows.shape[1]
    (w1, b1), (w2, b2), (wl, bl) = p["w1"], p["w2"], p["wl"]
    wu, bu = up
    c_out = wu.shape[1]
    wlp = _perm_wl(wl, C)
    body = functools.partial(_conv_pc_body, C=C, bs=bs)
    out = pl.pallas_call(
        body,
        grid=(BS // bs,),
        in_specs=[
            pl.BlockSpec((bs * K, D), lambda g: (g, 0)),
            pl.BlockSpec((bs, 3), lambda g: (g, 0)),
            pl.BlockSpec((bs, c_out), lambda g: (g, 0)),
            _full(w1.shape), _full(b1.shape), _full(w2.shape), _full(b2.shape),
            _full(wlp.shape), _full(bl.shape), _full(wu.shape), _full(bu.shape),
        ],
        out_specs=pl.BlockSpec((bs, c_out), lambda g: (g, 0)),
        out_shape=jax.ShapeDtypeStruct((BS, c_out), jnp.float32),
    )(rows, nx_flat, f1, w1, b1, w2, b2, wlp, bl, wu, bu)
    return out


def _head_body(xyz_ref, f_ref, w1x_ref, w1f_ref, b1_ref, w2_ref, b2_ref, o_ref):
    h = jax.nn.relu(
        jnp.dot(xyz_ref[0], w1x_ref[...], preferred_element_type=jnp.float32)
        + jnp.dot(f_ref[0], w1f_ref[...], preferred_element_type=jnp.float32)
        + b1_ref[...])
    h = jnp.dot(h, w2_ref[...], preferred_element_type=jnp.float32) + b2_ref[...]
    o_ref[0] = jnp.max(h, axis=0, keepdims=True)


def _head(xyz4, feats, gparams):
    B, S, Cf = feats.shape
    (W1, b1), (W2, b2) = gparams
    W1x, W1f = W1[:3], W1[3:]
    out = pl.pallas_call(
        _head_body,
        grid=(B,),
        in_specs=[
            pl.BlockSpec((1, S, 3), lambda b: (b, 0, 0)),
            pl.BlockSpec((1, S, Cf), lambda b: (b, 0, 0)),
            _full(W1x.shape), _full(W1f.shape), _full(b1.shape),
            _full(W2.shape), _full(b2.shape),
        ],
        out_specs=pl.BlockSpec((1, 1, W2.shape[1]), lambda b: (b, 0, 0)),
        out_shape=jax.ShapeDtypeStruct((B, 1, W2.shape[1]), jnp.float32),
    )(xyz4, feats, W1x, W1f, b1, W2, b2)
    return out


def kernel(xyz_0, xyz_1, xyz_2, xyz_3, xyz_4, normal_0, normal_1, normal_2, normal_3, normal_4, params):
    xyzs = [xyz_0, xyz_1, xyz_2, xyz_3, xyz_4]
    B = xyz_0.shape[0]
    feats = _stem(xyzs[0], params["pw"])

    for i, lp in enumerate(params["layers"]):
        xyz, new_xyz = xyzs[i], xyzs[i + 1]
        N, S = xyz.shape[1], new_xyz.shape[1]
        C = feats.shape[2]
        nx_flat = new_xyz.reshape(B * S, 3)

        idx = _knn(new_xyz, xyz, N)
        D1 = _round16(C + 3)
        tbl = _pad_cols(jnp.concatenate([feats, xyz], -1), D1).reshape(B * N, D1)
        rows = _sc_gather(tbl, idx.reshape(B * S * K))
        f1, hd = _conv_interp(rows, nx_flat, lp["interp"], lp["down"], C)

        nn_idx = _knn(new_xyz, new_xyz, S)
        mid = hd.shape[1]
        D2 = _round16(mid + 3)
        tbl2 = _pad_cols(
            jnp.concatenate([hd.reshape(B, S, mid), new_xyz], -1), D2
        ).reshape(B * S, D2)
        rows2 = _sc_gather(tbl2, nn_idx.reshape(B * S * K))
        out = _conv_pc(rows2, nx_flat, f1, lp["pc"], lp["up"], mid)
        feats = out.reshape(B, S, -1)

    local = feats
    g = _head(xyzs[-1], feats, params["global"])
    return local, g
